# Initial kernel scaffold; baseline (speedup 1.0000x reference)
import jax
import jax.numpy as jnp
from jax.experimental import pallas as pl


def kernel(x, edge_index, W1, b1, W2, b2, W3, b3, W4, b4, W5, b5, W6, b6, W_l2, b_l2, W_cls, b_cls):
    raise NotImplementedError("write your pallas kernel here")


# speedup vs baseline: 2.7871x; 2.7871x over previous
import functools

import jax
import jax.numpy as jnp
from jax import lax
from jax.experimental import pallas as pl
from jax.experimental.pallas import tpu as pltpu
from jax.experimental.pallas import tpu_sc as plsc

N = 10000
NP = 10240
E = 160000
D = 256
HALF = 128
NTILE = 16
CHUNK = 128
NCHUNK = 80
EP = NTILE * NCHUNK * CHUNK
PAD_NODE = 10200
RPT = NP // NTILE
BN = 1280

_f32 = jnp.float32


_mesh = plsc.VectorSubcoreMesh(core_axis_name="c", subcore_axis_name="s")


def _sc_degrees_body(src_h, dst_h, ones_h, z16_h, outd_h, ind_h, acc, idx_v, ones_v):
    cid = lax.axis_index("c")
    sid = lax.axis_index("s")
    base = sid * RPT
    pltpu.sync_copy(z16_h, acc.at[pl.ds(base, RPT)])

    @pl.when(cid == 0)
    def _():
        pltpu.sync_copy(src_h.at[sid], idx_v)

    @pl.when(cid == 1)
    def _():
        pltpu.sync_copy(dst_h.at[sid], idx_v)

    pltpu.sync_copy(ones_h, ones_v)
    plsc.subcore_barrier()

    def body(j, carry):
        pltpu.sync_copy(ones_v, acc.at[idx_v.at[j]], add=True)
        return carry

    lax.fori_loop(0, NCHUNK, body, 0)
    plsc.subcore_barrier()

    @pl.when(cid == 0)
    def _():
        pltpu.sync_copy(acc.at[pl.ds(base, RPT)], outd_h.at[pl.ds(base, RPT)])

    @pl.when(cid == 1)
    def _():
        pltpu.sync_copy(acc.at[pl.ds(base, RPT)], ind_h.at[pl.ds(base, RPT)])


def _sc_aggregate_body(hw0_h, hw1_h, src_h, dst_h, zrow_h, out0_h, out1_h,
                       acc, src_v, dst_v, gbuf, gsem):
    cid = lax.axis_index("c")
    sid = lax.axis_index("s")
    base = sid * RPT
    pltpu.sync_copy(src_h.at[sid], src_v)
    pltpu.sync_copy(dst_h.at[sid], dst_v)
    pltpu.sync_copy(zrow_h, acc.at[pl.ds(base, RPT)])
    plsc.subcore_barrier()

    def body(j, carry):
        @pl.when(cid == 0)
        def _():
            pltpu.async_copy(hw0_h.at[src_v.at[j]], gbuf, gsem).wait()

        @pl.when(cid == 1)
        def _():
            pltpu.async_copy(hw1_h.at[src_v.at[j]], gbuf, gsem).wait()

        pltpu.sync_copy(gbuf, acc.at[dst_v.at[j]], add=True)
        return carry

    lax.fori_loop(0, NCHUNK, body, 0)
    plsc.subcore_barrier()

    @pl.when(cid == 0)
    def _():
        pltpu.sync_copy(acc.at[pl.ds(base, RPT)], out0_h.at[pl.ds(base, RPT)])

    @pl.when(cid == 1)
    def _():
        pltpu.sync_copy(acc.at[pl.ds(base, RPT)], out1_h.at[pl.ds(base, RPT)])


_sc_degrees = pl.kernel(
    _sc_degrees_body,
    out_type=[
        jax.ShapeDtypeStruct((NP, HALF), _f32),
        jax.ShapeDtypeStruct((NP, HALF), _f32),
    ],
    mesh=_mesh,
    scratch_types=[
        pltpu.VMEM_SHARED((NP, HALF), _f32),
        pltpu.VMEM((NCHUNK, CHUNK), jnp.int32),
        pltpu.VMEM((CHUNK, HALF), _f32),
    ],
)

_sc_aggregate = pl.kernel(
    _sc_aggregate_body,
    out_type=[
        jax.ShapeDtypeStruct((NP, HALF), _f32),
        jax.ShapeDtypeStruct((NP, HALF), _f32),
    ],
    mesh=_mesh,
    scratch_types=[
        pltpu.VMEM_SHARED((NP, HALF), _f32),
        pltpu.VMEM((NCHUNK, CHUNK), jnp.int32),
        pltpu.VMEM((NCHUNK, CHUNK), jnp.int32),
        pltpu.VMEM((CHUNK, HALF), _f32),
        pltpu.SemaphoreType.DMA,
    ],
)


def _norm(deg_ref):
    return lax.rsqrt(jnp.maximum(deg_ref[:, 0:1], 1.0))


def _tc_first_body(x_ref, outd_ref, w_ref, hw0_ref, hw1_ref):
    out_n = _norm(outd_ref)
    y = jnp.dot(x_ref[...] * out_n, w_ref[...], preferred_element_type=_f32)
    hw0_ref[...] = y[:, :HALF]
    hw1_ref[...] = y[:, HALF:]


def _tc_mid_body(a0_ref, a1_ref, x_ref, ind_ref, outd_ref, wt_ref, wb_ref,
                 bp_ref, hw0_ref, hw1_ref):
    i = pl.program_id(0)
    in_n = _norm(ind_ref)
    out_n = _norm(outd_ref)
    agg = jnp.concatenate([a0_ref[...], a1_ref[...]], axis=1)
    h = jnp.maximum(agg * in_n + bp_ref[...], 0.0)
    y = (jnp.dot(h * out_n, wt_ref[...], preferred_element_type=_f32)
         + jnp.dot(x_ref[...] * out_n, wb_ref[...], preferred_element_type=_f32))
    ridx = i * BN + lax.broadcasted_iota(jnp.int32, (BN, 1), 0)
    y = jnp.where(ridx < N, y, 0.0)
    hw0_ref[...] = y[:, :HALF]
    hw1_ref[...] = y[:, HALF:]


def _tc_pool_body(a0_ref, a1_ref, x_ref, ind_ref, b6_ref, psum_ref, pmax_ref):
    i = pl.program_id(0)
    in_n = _norm(ind_ref)
    agg = jnp.concatenate([a0_ref[...], a1_ref[...]], axis=1)
    h = jnp.maximum(agg * in_n + b6_ref[...], 0.0)
    hcat = jnp.concatenate([h, x_ref[...]], axis=1)
    ridx = i * BN + lax.broadcasted_iota(jnp.int32, (BN, 1), 0)
    mask = ridx < N
    s = jnp.sum(jnp.where(mask, hcat, 0.0), axis=0, keepdims=True)
    m = jnp.max(jnp.where(mask, hcat, -jnp.inf), axis=0, keepdims=True)

    @pl.when(i == 0)
    def _():
        psum_ref[...] = s
        pmax_ref[...] = m

    @pl.when(i != 0)
    def _():
        psum_ref[...] = psum_ref[...] + s
        pmax_ref[...] = jnp.maximum(pmax_ref[...], m)


def _tc_head_body(psum_ref, pmax_ref, wa_ref, wb_ref, wc_ref, bl2_ref,
                  wcls_ref, bcls_ref, out_ref):
    mean = psum_ref[...] * (1.0 / N)
    z = (jnp.dot(mean, wa_ref[...], preferred_element_type=_f32)
         + jnp.dot(pmax_ref[...], wb_ref[...], preferred_element_type=_f32)
         + jnp.dot(psum_ref[...], wc_ref[...], preferred_element_type=_f32)
         + bl2_ref[...])
    z = jnp.maximum(z, 0.0)
    out_ref[...] = (jnp.dot(z, wcls_ref[...], preferred_element_type=_f32)
                    + bcls_ref[...])


def _row_spec(w):
    return pl.BlockSpec((BN, w), lambda i: (i, 0))


def _full_spec(shape):
    return pl.BlockSpec(shape, lambda i: (0, 0))


_tc_first = pl.pallas_call(
    _tc_first_body,
    grid=(NP // BN,),
    in_specs=[_row_spec(D), _row_spec(HALF), _full_spec((D, D))],
    out_specs=[_row_spec(HALF), _row_spec(HALF)],
    out_shape=[jax.ShapeDtypeStruct((NP, HALF), _f32)] * 2,
)

_tc_mid = pl.pallas_call(
    _tc_mid_body,
    grid=(NP // BN,),
    in_specs=[_row_spec(HALF), _row_spec(HALF), _row_spec(D), _row_spec(HALF),
              _row_spec(HALF), _full_spec((D, D)), _full_spec((D, D)),
              _full_spec((1, D))],
    out_specs=[_row_spec(HALF), _row_spec(HALF)],
    out_shape=[jax.ShapeDtypeStruct((NP, HALF), _f32)] * 2,
)

_tc_pool = pl.pallas_call(
    _tc_pool_body,
    grid=(NP // BN,),
    in_specs=[_row_spec(HALF), _row_spec(HALF), _row_spec(D), _row_spec(HALF),
              _full_spec((1, D))],
    out_specs=[_full_spec((1, 2 * D))] * 2,
    out_shape=[jax.ShapeDtypeStruct((1, 2 * D), _f32)] * 2,
)

_tc_head = pl.pallas_call(
    _tc_head_body,
    grid=(1,),
    in_specs=[_full_spec((1, 2 * D)), _full_spec((1, 2 * D)),
              _full_spec((2 * D, 6 * D)), _full_spec((2 * D, 6 * D)),
              _full_spec((2 * D, 6 * D)), _full_spec((1, 6 * D)),
              _full_spec((6 * D, 2)), _full_spec((1, 2))],
    out_specs=pl.BlockSpec((1, 2), lambda i: (0, 0)),
    out_shape=jax.ShapeDtypeStruct((1, 2), _f32),
)


def kernel(x, edge_index, W1, b1, W2, b2, W3, b3, W4, b4, W5, b5, W6, b6,
           W_l2, b_l2, W_cls, b_cls):
    xp = jnp.zeros((NP, D), _f32).at[:N].set(x)
    pad = jnp.full((EP - E,), PAD_NODE, jnp.int32)
    src3 = jnp.concatenate([edge_index[0], pad]).reshape(NTILE, NCHUNK, CHUNK)
    dst3 = jnp.concatenate([edge_index[1], pad]).reshape(NTILE, NCHUNK, CHUNK)
    zrow = jnp.zeros((RPT, HALF), _f32)
    z16 = jnp.zeros((RPT, HALF), _f32)
    ones16 = jnp.ones((CHUNK, HALF), _f32)

    outd16, ind16 = _sc_degrees(src3, dst3, ones16, z16)

    hw0, hw1 = _tc_first(xp, outd16, W1)
    a0, a1 = _sc_aggregate(hw0, hw1, src3, dst3, zrow)
    for (Wn, bp) in ((W2, b1), (W3, b2), (W4, b3), (W5, b4), (W6, b5)):
        hw0, hw1 = _tc_mid(a0, a1, xp, ind16, outd16, Wn[:D], Wn[D:],
                           bp.reshape(1, D))
        a0, a1 = _sc_aggregate(hw0, hw1, src3, dst3, zrow)

    psum, pmax = _tc_pool(a0, a1, xp, ind16, b6.reshape(1, D))
    return _tc_head(psum, pmax, W_l2[:2 * D], W_l2[2 * D:4 * D],
                    W_l2[4 * D:], b_l2.reshape(1, 6 * D), W_cls,
                    b_cls.reshape(1, 2))


# speedup vs baseline: 8.0510x; 2.8887x over previous
import functools

import jax
import jax.numpy as jnp
from jax import lax
from jax.experimental import pallas as pl
from jax.experimental.pallas import tpu as pltpu
from jax.experimental.pallas import tpu_sc as plsc

N = 10000
NP = 10240
E = 160000
D = 256
HALF = 128
NTILE = 16
CHUNK = 128
NCHUNK = 80
NWIN = NCHUNK // 2
EP = NTILE * NCHUNK * CHUNK
PAD_NODE = 10200
RPT = NP // NTILE
BN = 1280

_f32 = jnp.float32


_mesh = plsc.VectorSubcoreMesh(core_axis_name="c", subcore_axis_name="s")


def _sc_degrees_body(src_h, dst_h, ones_h, z16_h, outd_h, ind_h, acc, idx_v, ones_v):
    cid = lax.axis_index("c")
    sid = lax.axis_index("s")
    base = sid * RPT
    pltpu.sync_copy(z16_h, acc.at[pl.ds(base, RPT)])

    @pl.when(cid == 0)
    def _():
        pltpu.sync_copy(src_h.at[sid * 2], idx_v)

    @pl.when(cid == 1)
    def _():
        pltpu.sync_copy(dst_h.at[sid * 2], idx_v)

    pltpu.sync_copy(ones_h, ones_v)
    plsc.subcore_barrier()

    def body(j, carry):
        pltpu.sync_copy(ones_v, acc.at[idx_v.at[j]], add=True)
        return carry

    lax.fori_loop(0, NWIN, body, 0)

    @pl.when(cid == 0)
    def _():
        pltpu.sync_copy(src_h.at[sid * 2 + 1], idx_v)

    @pl.when(cid == 1)
    def _():
        pltpu.sync_copy(dst_h.at[sid * 2 + 1], idx_v)

    lax.fori_loop(0, NWIN, body, 0)
    plsc.subcore_barrier()

    @pl.when(cid == 0)
    def _():
        pltpu.sync_copy(acc.at[pl.ds(base, RPT)], outd_h.at[pl.ds(base, RPT)])

    @pl.when(cid == 1)
    def _():
        pltpu.sync_copy(acc.at[pl.ds(base, RPT)], ind_h.at[pl.ds(base, RPT)])


def _sc_aggregate_body(hw0_h, hw1_h, src_h, dst_h, zrow_h, out0_h, out1_h,
                       acc, src_v, dst_v, gbuf, gsem0, gsem1):
    cid = lax.axis_index("c")
    sid = lax.axis_index("s")
    base = sid * RPT
    pltpu.sync_copy(zrow_h, acc.at[pl.ds(base, RPT)])
    plsc.subcore_barrier()

    sems = (gsem0, gsem1)

    def fire(j, slot):
        @pl.when(cid == 0)
        def _():
            pltpu.make_async_copy(hw0_h.at[src_v.at[j]], gbuf.at[slot],
                                  sems[slot]).start()

        @pl.when(cid == 1)
        def _():
            pltpu.make_async_copy(hw1_h.at[src_v.at[j]], gbuf.at[slot],
                                  sems[slot]).start()

    def wait_g(j, slot):
        @pl.when(cid == 0)
        def _():
            pltpu.make_async_copy(hw0_h.at[src_v.at[j]], gbuf.at[slot],
                                  sems[slot]).wait()

        @pl.when(cid == 1)
        def _():
            pltpu.make_async_copy(hw1_h.at[src_v.at[j]], gbuf.at[slot],
                                  sems[slot]).wait()

    for h in range(2):
        pltpu.sync_copy(src_h.at[sid * 2 + h], src_v)
        pltpu.sync_copy(dst_h.at[sid * 2 + h], dst_v)
        fire(0, 0)

        def body(j2, carry):
            j = j2 * 2
            fire(j + 1, 1)
            wait_g(j, 0)
            pltpu.sync_copy(gbuf.at[0], acc.at[dst_v.at[j]], add=True)

            @pl.when(j + 2 < NWIN)
            def _():
                fire(j + 2, 0)

            wait_g(j + 1, 1)
            pltpu.sync_copy(gbuf.at[1], acc.at[dst_v.at[j + 1]], add=True)
            return carry

        lax.fori_loop(0, NWIN // 2, body, 0)

    plsc.subcore_barrier()

    @pl.when(cid == 0)
    def _():
        pltpu.sync_copy(acc.at[pl.ds(base, RPT)], out0_h.at[pl.ds(base, RPT)])

    @pl.when(cid == 1)
    def _():
        pltpu.sync_copy(acc.at[pl.ds(base, RPT)], out1_h.at[pl.ds(base, RPT)])


_sc_degrees = pl.kernel(
    _sc_degrees_body,
    out_type=[
        jax.ShapeDtypeStruct((NP, HALF), _f32),
        jax.ShapeDtypeStruct((NP, HALF), _f32),
    ],
    mesh=_mesh,
    scratch_types=[
        pltpu.VMEM_SHARED((NP, HALF), _f32),
        pltpu.VMEM((NWIN, CHUNK), jnp.int32),
        pltpu.VMEM((CHUNK, HALF), _f32),
    ],
)

_sc_aggregate = pl.kernel(
    _sc_aggregate_body,
    out_type=[
        jax.ShapeDtypeStruct((NP, HALF), _f32),
        jax.ShapeDtypeStruct((NP, HALF), _f32),
    ],
    mesh=_mesh,
    scratch_types=[
        pltpu.VMEM_SHARED((NP, HALF), _f32),
        pltpu.VMEM((NWIN, CHUNK), jnp.int32),
        pltpu.VMEM((NWIN, CHUNK), jnp.int32),
        pltpu.VMEM((2, CHUNK, HALF), _f32),
        pltpu.SemaphoreType.DMA,
        pltpu.SemaphoreType.DMA,
    ],
)


def _norm(deg_ref):
    return lax.rsqrt(jnp.maximum(deg_ref[:, 0:1], 1.0))


def _tc_first_body(x_ref, outd_ref, w_ref, hw0_ref, hw1_ref):
    out_n = _norm(outd_ref)
    y = jnp.dot(x_ref[...] * out_n, w_ref[...], preferred_element_type=_f32)
    hw0_ref[...] = y[:, :HALF]
    hw1_ref[...] = y[:, HALF:]


def _tc_mid_body(a0_ref, a1_ref, x_ref, ind_ref, outd_ref, wt_ref, wb_ref,
                 bp_ref, hw0_ref, hw1_ref):
    i = pl.program_id(0)
    in_n = _norm(ind_ref)
    out_n = _norm(outd_ref)
    agg = jnp.concatenate([a0_ref[...], a1_ref[...]], axis=1)
    h = jnp.maximum(agg * in_n + bp_ref[...], 0.0)
    y = (jnp.dot(h * out_n, wt_ref[...], preferred_element_type=_f32)
         + jnp.dot(x_ref[...] * out_n, wb_ref[...], preferred_element_type=_f32))
    ridx = i * BN + lax.broadcasted_iota(jnp.int32, (BN, 1), 0)
    y = jnp.where(ridx < N, y, 0.0)
    hw0_ref[...] = y[:, :HALF]
    hw1_ref[...] = y[:, HALF:]


def _tc_pool_body(a0_ref, a1_ref, x_ref, ind_ref, b6_ref, psum_ref, pmax_ref):
    i = pl.program_id(0)
    in_n = _norm(ind_ref)
    agg = jnp.concatenate([a0_ref[...], a1_ref[...]], axis=1)
    h = jnp.maximum(agg * in_n + b6_ref[...], 0.0)
    hcat = jnp.concatenate([h, x_ref[...]], axis=1)
    ridx = i * BN + lax.broadcasted_iota(jnp.int32, (BN, 1), 0)
    mask = ridx < N
    s = jnp.sum(jnp.where(mask, hcat, 0.0), axis=0, keepdims=True)
    m = jnp.max(jnp.where(mask, hcat, -jnp.inf), axis=0, keepdims=True)

    @pl.when(i == 0)
    def _():
        psum_ref[...] = s
        pmax_ref[...] = m

    @pl.when(i != 0)
    def _():
        psum_ref[...] = psum_ref[...] + s
        pmax_ref[...] = jnp.maximum(pmax_ref[...], m)


def _tc_head_body(psum_ref, pmax_ref, wa_ref, wb_ref, wc_ref, bl2_ref,
                  wcls_ref, bcls_ref, out_ref):
    mean = psum_ref[...] * (1.0 / N)
    z = (jnp.dot(mean, wa_ref[...], preferred_element_type=_f32)
         + jnp.dot(pmax_ref[...], wb_ref[...], preferred_element_type=_f32)
         + jnp.dot(psum_ref[...], wc_ref[...], preferred_element_type=_f32)
         + bl2_ref[...])
    z = jnp.maximum(z, 0.0)
    out_ref[...] = (jnp.dot(z, wcls_ref[...], preferred_element_type=_f32)
                    + bcls_ref[...])


def _row_spec(w):
    return pl.BlockSpec((BN, w), lambda i: (i, 0))


def _full_spec(shape):
    return pl.BlockSpec(shape, lambda i: (0, 0))


_tc_first = pl.pallas_call(
    _tc_first_body,
    grid=(NP // BN,),
    in_specs=[_row_spec(D), _row_spec(HALF), _full_spec((D, D))],
    out_specs=[_row_spec(HALF), _row_spec(HALF)],
    out_shape=[jax.ShapeDtypeStruct((NP, HALF), _f32)] * 2,
)

_tc_mid = pl.pallas_call(
    _tc_mid_body,
    grid=(NP // BN,),
    in_specs=[_row_spec(HALF), _row_spec(HALF), _row_spec(D), _row_spec(HALF),
              _row_spec(HALF), _full_spec((D, D)), _full_spec((D, D)),
              _full_spec((1, D))],
    out_specs=[_row_spec(HALF), _row_spec(HALF)],
    out_shape=[jax.ShapeDtypeStruct((NP, HALF), _f32)] * 2,
)

_tc_pool = pl.pallas_call(
    _tc_pool_body,
    grid=(NP // BN,),
    in_specs=[_row_spec(HALF), _row_spec(HALF), _row_spec(D), _row_spec(HALF),
              _full_spec((1, D))],
    out_specs=[_full_spec((1, 2 * D))] * 2,
    out_shape=[jax.ShapeDtypeStruct((1, 2 * D), _f32)] * 2,
)

_tc_head = pl.pallas_call(
    _tc_head_body,
    grid=(1,),
    in_specs=[_full_spec((1, 2 * D)), _full_spec((1, 2 * D)),
              _full_spec((2 * D, 6 * D)), _full_spec((2 * D, 6 * D)),
              _full_spec((2 * D, 6 * D)), _full_spec((1, 6 * D)),
              _full_spec((6 * D, 2)), _full_spec((1, 2))],
    out_specs=pl.BlockSpec((1, 2), lambda i: (0, 0)),
    out_shape=jax.ShapeDtypeStruct((1, 2), _f32),
)


def kernel(x, edge_index, W1, b1, W2, b2, W3, b3, W4, b4, W5, b5, W6, b6,
           W_l2, b_l2, W_cls, b_cls):
    xp = jnp.zeros((NP, D), _f32).at[:N].set(x)
    pad = N + jnp.arange(EP - E, dtype=jnp.int32) % (NP - N)
    src3 = jnp.concatenate([edge_index[0], pad]).reshape(2 * NTILE, NWIN, CHUNK)
    dst3 = jnp.concatenate([edge_index[1], pad]).reshape(2 * NTILE, NWIN, CHUNK)
    zrow = jnp.zeros((RPT, HALF), _f32)
    z16 = jnp.zeros((RPT, HALF), _f32)
    ones16 = jnp.ones((CHUNK, HALF), _f32)

    outd16, ind16 = _sc_degrees(src3, dst3, ones16, z16)

    hw0, hw1 = _tc_first(xp, outd16, W1)
    a0, a1 = _sc_aggregate(hw0, hw1, src3, dst3, zrow)
    for (Wn, bp) in ((W2, b1), (W3, b2), (W4, b3), (W5, b4), (W6, b5)):
        hw0, hw1 = _tc_mid(a0, a1, xp, ind16, outd16, Wn[:D], Wn[D:],
                           bp.reshape(1, D))
        a0, a1 = _sc_aggregate(hw0, hw1, src3, dst3, zrow)

    psum, pmax = _tc_pool(a0, a1, xp, ind16, b6.reshape(1, D))
    return _tc_head(psum, pmax, W_l2[:2 * D], W_l2[2 * D:4 * D],
                    W_l2[4 * D:], b_l2.reshape(1, 6 * D), W_cls,
                    b_cls.reshape(1, 2))
